# baseline (speedup 1.0000x reference)
import functools

import jax
import jax.numpy as jnp
from jax import lax
from jax.experimental import pallas as pl
from jax.experimental.pallas import tpu as pltpu
from jax.experimental.pallas import tpu_sc as plsc

NC = 2
NS = 16
NW = NC * NS
W = 128
H = 128
FC = 32
NCHUNK = H // FC
CW = 16
BLK = 512
NBUF = 4
PADR = NW + NBUF + 4


def _round_up(a, m):
    return (a + m - 1) // m * m


def _sc_mesh():
    return plsc.VectorSubcoreMesh(core_axis_name="c", subcore_axis_name="s")


def _sc_agg_layer(tables_list, ek_meta, zeros, max_pad):
    n_ek = len(ek_meta)
    n_nt = len(tables_list)
    out_types = tuple(
        jax.ShapeDtypeStruct((NCHUNK, m["n_dst_pad"], FC), jnp.float32)
        for m in ek_meta)
    inputs = list(tables_list)
    for m in ek_meta:
        inputs += [m["srcA"], m["dst2"]]
    inputs.append(zeros)

    @functools.partial(
        pl.kernel,
        mesh=_sc_mesh(),
        out_type=out_types,
        compiler_params=pltpu.CompilerParams(use_tc_tiling_on_sc=False),
        scratch_types=[
            pltpu.VMEM((NBUF, W), jnp.int32),
            pltpu.VMEM((NBUF, W), jnp.int32),
            pltpu.VMEM((NBUF, W), jnp.int32),
            pltpu.VMEM((NBUF, W), jnp.int32),
            [pltpu.VMEM((W, FC), jnp.float32) for _ in range(NBUF)],
            pltpu.VMEM_SHARED((max_pad, FC), jnp.float32),
            [pltpu.SemaphoreType.DMA for _ in range(NBUF)],
            [pltpu.SemaphoreType.DMA for _ in range(NBUF)],
            pltpu.SemaphoreType.DMA,
            pltpu.SemaphoreType.DMA,
        ],
    )
    def k(*args):
        nt_refs = args[:n_nt]
        ek_refs = [(args[n_nt + 2 * i], args[n_nt + 2 * i + 1])
                   for i in range(n_ek)]
        zero_h = args[n_nt + 2 * n_ek]
        out_refs = args[n_nt + 2 * n_ek + 1:n_nt + 2 * n_ek + 1 + n_ek]
        (st_as, st_ad, st_bs, st_bd, rows, acc, g_sem, s_sem, sem_a,
         sem_b) = args[n_nt + 2 * n_ek + 1 + n_ek:]
        cid = lax.axis_index("c")
        sid = lax.axis_index("s")
        stages = ((st_as, st_ad, sem_a), (st_bs, st_bd, sem_b))

        def phase(table_h, srcA_h, dst_h, out_h, nwin, n_dst_pad, sub):
            n_grp = -(-nwin // NBUF)
            n_sup = n_grp // 2
            rpt = n_dst_pad // NS
            r0 = sid * rpt
            chunk = 2 * cid + sub
            w0 = sid * nwin

            def stage_load(g, st):
                pltpu.async_copy(
                    srcA_h.at[chunk, pl.ds(w0 + g * NBUF, NBUF)], st[0],
                    st[2])
                pltpu.async_copy(
                    dst_h.at[pl.ds(w0 + g * NBUF, NBUF)], st[1], st[2])

            def stage_wait(g, st):
                pltpu.make_async_copy(
                    srcA_h.at[chunk, pl.ds(w0 + g * NBUF, NBUF)], st[0],
                    st[2]).wait()
                pltpu.make_async_copy(
                    dst_h.at[pl.ds(w0 + g * NBUF, NBUF)], st[1],
                    st[2]).wait()

            def g_start(b, st):
                pltpu.async_copy(table_h.at[st[0].at[b]], rows[b], g_sem[b])

            def g_wait(b, st):
                pltpu.make_async_copy(table_h.at[st[0].at[b]], rows[b],
                                      g_sem[b]).wait()

            def s_start(b, st):
                pltpu.async_copy(rows[b], acc.at[st[1].at[b]], s_sem[b],
                                 add=True)

            def s_wait(b, st):
                pltpu.make_async_copy(rows[b], acc.at[st[1].at[b]],
                                      s_sem[b]).wait()

            pltpu.sync_copy(zero_h.at[pl.ds(r0, rpt)], acc.at[pl.ds(r0, rpt)])
            plsc.subcore_barrier()

            stage_load(0, stages[0])
            stage_wait(0, stages[0])
            for b in range(min(NBUF, nwin)):
                g_start(b, stages[0])

            def grp(gprev, gnext, cst, nst):
                for b in range(NBUF):
                    @pl.when(gprev * NBUF + b < nwin)
                    def _(b=b):
                        g_wait(b, cst)
                        s_start(b, cst)

                @pl.when(gnext * NBUF < nwin)
                def _():
                    stage_wait(gnext, nst)

                for b in range(NBUF):
                    @pl.when(gnext * NBUF + b < nwin)
                    def _(b=b):
                        s_wait(b, cst)
                        g_start(b, nst)

            @pl.loop(0, n_sup)
            def _sup(gg):
                g0 = 2 * gg
                g1 = g0 + 1

                @pl.when(g1 * NBUF < nwin)
                def _():
                    stage_load(g1, stages[1])

                grp(g0, g1, stages[0], stages[1])

                @pl.when((g1 + 1) * NBUF < nwin)
                def _():
                    stage_load(g1 + 1, stages[0])

                grp(g1, g1 + 1, stages[1], stages[0])

            if n_grp % 2 == 1:
                gl = n_grp - 1
                lst = stages[gl % 2]
                for b in range(NBUF):
                    if gl * NBUF + b < nwin:
                        g_wait(b, lst)
                        s_start(b, lst)
            for b in range(min(NBUF, nwin)):
                s_wait(b, stages[0])

            plsc.subcore_barrier()
            pltpu.sync_copy(acc.at[pl.ds(r0, rpt)],
                            out_h.at[chunk, pl.ds(r0, rpt)])
            plsc.subcore_barrier()

        for sub in range(2):
            for i, m in enumerate(ek_meta):
                phase(nt_refs[m["tbl"]], ek_refs[i][0], ek_refs[i][1],
                      out_refs[i], m["rows"] // NS, m["n_dst_pad"], sub)

    return k(*inputs)


def _sc_count_merged(ek_meta, zeros16, ones, max_pad):
    n_ek = len(ek_meta)
    out_types = tuple(
        jax.ShapeDtypeStruct((NC, m["n_dst_pad"], CW), jnp.float32)
        for m in ek_meta)
    inputs = [m["dst2"] for m in ek_meta] + [zeros16, ones]

    @functools.partial(
        pl.kernel,
        mesh=_sc_mesh(),
        out_type=out_types,
        compiler_params=pltpu.CompilerParams(use_tc_tiling_on_sc=False),
        scratch_types=[
            pltpu.VMEM((W,), jnp.int32),
            pltpu.VMEM((W, CW), jnp.float32),
            pltpu.VMEM_SHARED((max_pad, CW), jnp.float32),
            pltpu.SemaphoreType.DMA,
        ],
    )
    def k(*args):
        dst_refs = args[:n_ek]
        zero_h, ones_h = args[n_ek], args[n_ek + 1]
        out_refs = args[n_ek + 2:2 * n_ek + 2]
        idx_d, ones_v, acc, sem = args[2 * n_ek + 2:]
        cid = lax.axis_index("c")
        sid = lax.axis_index("s")
        wid = sid * NC + cid
        pltpu.sync_copy(ones_h, ones_v)

        for i, m in enumerate(ek_meta):
            dst_h, out_h = dst_refs[i], out_refs[i]
            rows = m["rows"]
            nwin = -(-rows // NW)
            rpt = m["n_dst_pad"] // NS
            r0 = sid * rpt
            pltpu.sync_copy(zero_h.at[pl.ds(r0, rpt)], acc.at[pl.ds(r0, rpt)])
            plsc.subcore_barrier()

            @pl.loop(0, nwin)
            def _win(win, dst_h=dst_h, nwin=nwin, rows=rows):
                w = wid * nwin + win

                @pl.when(w < rows)
                def _():
                    pltpu.sync_copy(dst_h.at[w], idx_d)
                    pltpu.sync_copy(ones_v, acc.at[idx_d], add=True)

            plsc.subcore_barrier()
            pltpu.sync_copy(acc.at[pl.ds(r0, rpt)],
                            out_h.at[cid, pl.ds(r0, rpt)])
            plsc.subcore_barrier()

    return k(*inputs)


def _layernorm_relu(e, g, b, relu):
    mu = jnp.mean(e, axis=1, keepdims=True)
    var = jnp.mean(jnp.square(e - mu), axis=1, keepdims=True)
    out = (e - mu) * lax.rsqrt(var + 1e-5) * g + b
    return jnp.maximum(out, 0.0) if relu else out


def _tc_encoder(xin, p, lnp):
    n, d = xin.shape
    grid = pl.cdiv(n, BLK)

    def body(x_r, w1_r, b1_r, w2_r, b2_r, g_r, bb_r, o_r, oc_r):
        h1 = jnp.dot(x_r[...], w1_r[...], preferred_element_type=jnp.float32)
        h1 = jnp.maximum(h1 + b1_r[...], 0.0)
        e = jnp.dot(h1, w2_r[...], preferred_element_type=jnp.float32)
        e = e + b2_r[...]
        hh = _layernorm_relu(e, g_r[...], bb_r[...], relu=False)
        o_r[...] = hh
        for c in range(NCHUNK):
            oc_r[c] = hh[:, c * FC:(c + 1) * FC]

    return pl.pallas_call(
        body,
        grid=(grid,),
        in_specs=[
            pl.BlockSpec((BLK, d), lambda i: (i, 0)),
            pl.BlockSpec((d, H), lambda i: (0, 0)),
            pl.BlockSpec((1, H), lambda i: (0, 0)),
            pl.BlockSpec((H, H), lambda i: (0, 0)),
            pl.BlockSpec((1, H), lambda i: (0, 0)),
            pl.BlockSpec((1, H), lambda i: (0, 0)),
            pl.BlockSpec((1, H), lambda i: (0, 0)),
        ],
        out_specs=[
            pl.BlockSpec((BLK, H), lambda i: (i, 0)),
            pl.BlockSpec((NCHUNK, BLK, FC), lambda i: (0, i, 0)),
        ],
        out_shape=[
            jax.ShapeDtypeStruct((n, H), jnp.float32),
            jax.ShapeDtypeStruct((NCHUNK, n, FC), jnp.float32),
        ],
    )(xin, p["W1"], p["b1"].reshape(1, H), p["W2"], p["b2"].reshape(1, H),
      lnp["g"].reshape(1, H), lnp["b"].reshape(1, H))


def _tc_update(h, aggs, cnts, wls, wr_sum_eye, bias_sum, g, b, want_table):
    n = h.shape[0]
    k_num = len(aggs)
    grid = pl.cdiv(n, BLK)

    def body(*refs):
        h_r = refs[0]
        agg_rs = refs[1:1 + k_num]
        cnt_rs = refs[1 + k_num:1 + 2 * k_num]
        wl_r = refs[1 + 2 * k_num]
        wr_r, bias_r, g_r, b_r = refs[2 + 2 * k_num:6 + 2 * k_num]
        o_rs = refs[6 + 2 * k_num:]
        u = jnp.dot(h_r[...], wr_r[...], preferred_element_type=jnp.float32)
        u = u + bias_r[...]
        means = []
        for k in range(k_num):
            a = agg_rs[k]
            c = cnt_rs[k]
            cs = c[0, :, 0:1] + c[1, :, 0:1]
            inv = 1.0 / jnp.maximum(cs, 1.0)
            mk = jnp.concatenate([a[ch] for ch in range(NCHUNK)], axis=1)
            means.append(mk * inv)
        mcat = jnp.concatenate(means, axis=1) if k_num > 1 else means[0]
        u = u + jnp.dot(mcat, wl_r[...], preferred_element_type=jnp.float32)
        hh = _layernorm_relu(u, g_r[...], b_r[...], relu=True)
        o_rs[0][...] = hh
        if want_table:
            for ch in range(NCHUNK):
                o_rs[1][ch] = hh[:, ch * FC:(ch + 1) * FC]

    in_specs = [pl.BlockSpec((BLK, H), lambda i: (i, 0))]
    in_specs += [pl.BlockSpec((NCHUNK, BLK, FC), lambda i: (0, i, 0))
                 for _ in range(k_num)]
    in_specs += [pl.BlockSpec((NC, BLK, CW), lambda i: (0, i, 0))
                 for _ in range(k_num)]
    in_specs += [
        pl.BlockSpec((k_num * H, H), lambda i: (0, 0)),
        pl.BlockSpec((H, H), lambda i: (0, 0)),
        pl.BlockSpec((1, H), lambda i: (0, 0)),
        pl.BlockSpec((1, H), lambda i: (0, 0)),
        pl.BlockSpec((1, H), lambda i: (0, 0)),
    ]
    out_specs = [pl.BlockSpec((BLK, H), lambda i: (i, 0))]
    out_shape = [jax.ShapeDtypeStruct((n, H), jnp.float32)]
    if want_table:
        out_specs.append(pl.BlockSpec((NCHUNK, BLK, FC), lambda i: (0, i, 0)))
        out_shape.append(jax.ShapeDtypeStruct((NCHUNK, n, FC), jnp.float32))

    wl_cat = jnp.concatenate(wls, axis=0) if k_num > 1 else wls[0]
    outs = pl.pallas_call(
        body, grid=(grid,), in_specs=in_specs, out_specs=out_specs,
        out_shape=out_shape,
    )(h, *aggs, *cnts, wl_cat, wr_sum_eye, bias_sum.reshape(1, H),
      g.reshape(1, H), b.reshape(1, H))
    return (outs[0], outs[1]) if want_table else (outs[0], None)


def kernel(x, params, edges):
    node_types = list(x.keys())
    nt_index = {nt: i for i, nt in enumerate(node_types)}
    n_nodes = {nt: x[nt].shape[0] for nt in node_types}
    n_pad = {nt: _round_up(n_nodes[nt], NS * 8) for nt in node_types}
    max_pad = max(n_pad.values())

    h, tables = {}, {}
    for nt in node_types:
        h[nt], tab = _tc_encoder(x[nt], params["enc"][nt],
                                 params["enc_ln"][nt])
        tables[nt] = tab.reshape(NCHUNK * n_nodes[nt], FC)

    ek_meta = []
    ek_names = []
    for ek, ei in edges.items():
        src_t, _, dst_t = ek.split("__")
        e_num = ei.shape[1]
        e_pad = _round_up(e_num, NS * W)
        rows = e_pad // W
        src = jnp.concatenate(
            [ei[0], jnp.zeros((e_pad + PADR * W - e_num,), jnp.int32)])
        dst = jnp.concatenate(
            [ei[1],
             jnp.full((e_pad + PADR * W - e_num,), n_nodes[dst_t],
                      jnp.int32)])
        offs = (jnp.arange(NCHUNK, dtype=jnp.int32) * n_nodes[src_t])
        srcA = (src[None, :] + offs[:, None]).reshape(NCHUNK, rows + PADR, W)
        dst2 = dst.reshape(rows + PADR, W)
        ek_meta.append({
            "srcA": srcA, "dst2": dst2, "rows": rows,
            "n_dst_pad": n_pad[dst_t], "tbl": nt_index[src_t],
            "src_t": src_t, "dst_t": dst_t,
        })
        ek_names.append(ek)

    zeros_f = jnp.zeros((max_pad, FC), jnp.float32)
    zeros_c = jnp.zeros((max_pad, CW), jnp.float32)
    ones = jnp.ones((W, CW), jnp.float32)
    cnts = _sc_count_merged(ek_meta, zeros_c, ones, max_pad)

    for li, layer in enumerate(params["layers"]):
        want_table = li + 1 < len(params["layers"])
        tables_list = [tables[nt] for nt in node_types]
        aggs = _sc_agg_layer(tables_list, ek_meta, zeros_f, max_pad)
        gathered = {nt: [] for nt in node_types}
        for i, ek in enumerate(ek_names):
            dst_t = ek_meta[i]["dst_t"]
            gathered[dst_t].append((aggs[i], cnts[i], layer["conv"][ek]))
        new_h, new_tables = {}, {}
        for nt in node_types:
            parts = gathered[nt]
            ag = [p[0] for p in parts]
            cn = [p[1] for p in parts]
            wls = [p[2]["Wl"] for p in parts]
            wr_sum = sum(p[2]["Wr"] for p in parts) + jnp.eye(
                H, dtype=jnp.float32)
            bias_sum = sum(p[2]["bl"] for p in parts)
            ln = layer["ln"][nt]
            new_h[nt], tab = _tc_update(h[nt], ag, cn, wls, wr_sum,
                                        bias_sum, ln["g"], ln["b"],
                                        want_table)
            if want_table:
                new_tables[nt] = tab.reshape(NCHUNK * n_nodes[nt], FC)
        h = new_h
        if want_table:
            tables = new_tables

    return (h["note"], h["chord"], h["onset"])


# speedup vs baseline: 1.3153x; 1.3153x over previous
import functools

import jax
import jax.numpy as jnp
from jax import lax
from jax.experimental import pallas as pl
from jax.experimental.pallas import tpu as pltpu
from jax.experimental.pallas import tpu_sc as plsc

NC = 2
NS = 16
NW = NC * NS
W = 128
H = 128
FC = 32
NCHUNK = H // FC
CW = 16
BLK = 512
NBUF = 4
PADR = NW + NBUF + 4


def _round_up(a, m):
    return (a + m - 1) // m * m


def _sc_mesh():
    return plsc.VectorSubcoreMesh(core_axis_name="c", subcore_axis_name="s")


def _sc_agg_layer(tables_list, ek_meta, zeros, max_pad):
    n_ek = len(ek_meta)
    n_nt = len(tables_list)
    out_types = tuple(
        jax.ShapeDtypeStruct((m["n_dst_pad"], H), jnp.float32)
        for m in ek_meta)
    inputs = list(tables_list)
    for m in ek_meta:
        inputs += [m["srcA"], m["dst2"]]
    inputs.append(zeros)

    @functools.partial(
        pl.kernel,
        mesh=_sc_mesh(),
        out_type=out_types,
        compiler_params=pltpu.CompilerParams(use_tc_tiling_on_sc=False),
        scratch_types=[
            pltpu.VMEM((NBUF, W), jnp.int32),
            pltpu.VMEM((NBUF, W), jnp.int32),
            pltpu.VMEM((NBUF, W), jnp.int32),
            pltpu.VMEM((NBUF, W), jnp.int32),
            [pltpu.VMEM((W, FC), jnp.float32) for _ in range(NBUF)],
            pltpu.VMEM_SHARED((max_pad, FC), jnp.float32),
            [pltpu.SemaphoreType.DMA for _ in range(NBUF)],
            [pltpu.SemaphoreType.DMA for _ in range(NBUF)],
            pltpu.SemaphoreType.DMA,
            pltpu.SemaphoreType.DMA,
        ],
    )
    def k(*args):
        nt_refs = args[:n_nt]
        ek_refs = [(args[n_nt + 2 * i], args[n_nt + 2 * i + 1])
                   for i in range(n_ek)]
        zero_h = args[n_nt + 2 * n_ek]
        out_refs = args[n_nt + 2 * n_ek + 1:n_nt + 2 * n_ek + 1 + n_ek]
        (st_as, st_ad, st_bs, st_bd, rows, acc, g_sem, s_sem, sem_a,
         sem_b) = args[n_nt + 2 * n_ek + 1 + n_ek:]
        cid = lax.axis_index("c")
        sid = lax.axis_index("s")
        stages = ((st_as, st_ad, sem_a), (st_bs, st_bd, sem_b))

        def phase(table_h, srcA_h, dst_h, out_h, nwin, n_dst_pad, sub):
            n_grp = -(-nwin // NBUF)
            n_sup = n_grp // 2
            rpt = n_dst_pad // NS
            r0 = sid * rpt
            chunk = 2 * cid + sub
            w0 = sid * nwin

            def stage_load(g, st):
                pltpu.async_copy(
                    srcA_h.at[chunk, pl.ds(w0 + g * NBUF, NBUF)], st[0],
                    st[2])
                pltpu.async_copy(
                    dst_h.at[pl.ds(w0 + g * NBUF, NBUF)], st[1], st[2])

            def stage_wait(g, st):
                pltpu.make_async_copy(
                    srcA_h.at[chunk, pl.ds(w0 + g * NBUF, NBUF)], st[0],
                    st[2]).wait()
                pltpu.make_async_copy(
                    dst_h.at[pl.ds(w0 + g * NBUF, NBUF)], st[1],
                    st[2]).wait()

            def g_start(b, st):
                pltpu.async_copy(table_h.at[st[0].at[b]], rows[b], g_sem[b])

            def g_wait(b, st):
                pltpu.make_async_copy(table_h.at[st[0].at[b]], rows[b],
                                      g_sem[b]).wait()

            def s_start(b, st):
                pltpu.async_copy(rows[b], acc.at[st[1].at[b]], s_sem[b],
                                 add=True)

            def s_wait(b, st):
                pltpu.make_async_copy(rows[b], acc.at[st[1].at[b]],
                                      s_sem[b]).wait()

            pltpu.sync_copy(zero_h.at[pl.ds(r0, rpt)], acc.at[pl.ds(r0, rpt)])
            plsc.subcore_barrier()

            stage_load(0, stages[0])
            stage_wait(0, stages[0])
            for b in range(min(NBUF, nwin)):
                g_start(b, stages[0])

            def grp(gprev, gnext, cst, nst):
                for b in range(NBUF):
                    @pl.when(gprev * NBUF + b < nwin)
                    def _(b=b):
                        g_wait(b, cst)
                        s_start(b, cst)

                @pl.when(gnext * NBUF < nwin)
                def _():
                    stage_wait(gnext, nst)

                for b in range(NBUF):
                    @pl.when(gnext * NBUF + b < nwin)
                    def _(b=b):
                        s_wait(b, cst)
                        g_start(b, nst)

            @pl.loop(0, n_sup)
            def _sup(gg):
                g0 = 2 * gg
                g1 = g0 + 1

                @pl.when(g1 * NBUF < nwin)
                def _():
                    stage_load(g1, stages[1])

                grp(g0, g1, stages[0], stages[1])

                @pl.when((g1 + 1) * NBUF < nwin)
                def _():
                    stage_load(g1 + 1, stages[0])

                grp(g1, g1 + 1, stages[1], stages[0])

            if n_grp % 2 == 1:
                gl = n_grp - 1
                lst = stages[gl % 2]
                for b in range(NBUF):
                    if gl * NBUF + b < nwin:
                        g_wait(b, lst)
                        s_start(b, lst)
            for b in range(min(NBUF, nwin)):
                s_wait(b, stages[0])

            plsc.subcore_barrier()
            pltpu.sync_copy(acc.at[pl.ds(r0, rpt)],
                            out_h.at[pl.ds(r0, rpt), pl.ds(chunk * FC, FC)])
            plsc.subcore_barrier()

        for sub in range(2):
            for i, m in enumerate(ek_meta):
                phase(nt_refs[m["tbl"]], ek_refs[i][0], ek_refs[i][1],
                      out_refs[i], m["rows"] // NS, m["n_dst_pad"], sub)

    return k(*inputs)


def _sc_count_merged(ek_meta, zeros16, ones, max_pad):
    n_ek = len(ek_meta)
    out_types = tuple(
        jax.ShapeDtypeStruct((NC, m["n_dst_pad"], CW), jnp.float32)
        for m in ek_meta)
    inputs = [m["dst2"] for m in ek_meta] + [zeros16, ones]

    @functools.partial(
        pl.kernel,
        mesh=_sc_mesh(),
        out_type=out_types,
        compiler_params=pltpu.CompilerParams(use_tc_tiling_on_sc=False),
        scratch_types=[
            pltpu.VMEM((W,), jnp.int32),
            pltpu.VMEM((W, CW), jnp.float32),
            pltpu.VMEM_SHARED((max_pad, CW), jnp.float32),
            pltpu.SemaphoreType.DMA,
        ],
    )
    def k(*args):
        dst_refs = args[:n_ek]
        zero_h, ones_h = args[n_ek], args[n_ek + 1]
        out_refs = args[n_ek + 2:2 * n_ek + 2]
        idx_d, ones_v, acc, sem = args[2 * n_ek + 2:]
        cid = lax.axis_index("c")
        sid = lax.axis_index("s")
        wid = sid * NC + cid
        pltpu.sync_copy(ones_h, ones_v)

        for i, m in enumerate(ek_meta):
            dst_h, out_h = dst_refs[i], out_refs[i]
            rows = m["rows"]
            nwin = -(-rows // NW)
            rpt = m["n_dst_pad"] // NS
            r0 = sid * rpt
            pltpu.sync_copy(zero_h.at[pl.ds(r0, rpt)], acc.at[pl.ds(r0, rpt)])
            plsc.subcore_barrier()

            @pl.loop(0, nwin)
            def _win(win, dst_h=dst_h, nwin=nwin, rows=rows):
                w = wid * nwin + win

                @pl.when(w < rows)
                def _():
                    pltpu.sync_copy(dst_h.at[w], idx_d)
                    pltpu.sync_copy(ones_v, acc.at[idx_d], add=True)

            plsc.subcore_barrier()
            pltpu.sync_copy(acc.at[pl.ds(r0, rpt)],
                            out_h.at[cid, pl.ds(r0, rpt)])
            plsc.subcore_barrier()

    return k(*inputs)


def _layernorm_relu(e, g, b, relu):
    mu = jnp.mean(e, axis=1, keepdims=True)
    var = jnp.mean(jnp.square(e - mu), axis=1, keepdims=True)
    out = (e - mu) * lax.rsqrt(var + 1e-5) * g + b
    return jnp.maximum(out, 0.0) if relu else out


def _tc_encoder(xin, p, lnp):
    n, d = xin.shape
    grid = pl.cdiv(n, BLK)

    def body(x_r, w1_r, b1_r, w2_r, b2_r, g_r, bb_r, o_r, oc_r):
        h1 = jnp.dot(x_r[...], w1_r[...], preferred_element_type=jnp.float32)
        h1 = jnp.maximum(h1 + b1_r[...], 0.0)
        e = jnp.dot(h1, w2_r[...], preferred_element_type=jnp.float32)
        e = e + b2_r[...]
        hh = _layernorm_relu(e, g_r[...], bb_r[...], relu=False)
        o_r[...] = hh
        for c in range(NCHUNK):
            oc_r[c] = hh[:, c * FC:(c + 1) * FC]

    return pl.pallas_call(
        body,
        grid=(grid,),
        in_specs=[
            pl.BlockSpec((BLK, d), lambda i: (i, 0)),
            pl.BlockSpec((d, H), lambda i: (0, 0)),
            pl.BlockSpec((1, H), lambda i: (0, 0)),
            pl.BlockSpec((H, H), lambda i: (0, 0)),
            pl.BlockSpec((1, H), lambda i: (0, 0)),
            pl.BlockSpec((1, H), lambda i: (0, 0)),
            pl.BlockSpec((1, H), lambda i: (0, 0)),
        ],
        out_specs=[
            pl.BlockSpec((BLK, H), lambda i: (i, 0)),
            pl.BlockSpec((NCHUNK, BLK, FC), lambda i: (0, i, 0)),
        ],
        out_shape=[
            jax.ShapeDtypeStruct((n, H), jnp.float32),
            jax.ShapeDtypeStruct((NCHUNK, n, FC), jnp.float32),
        ],
    )(xin, p["W1"], p["b1"].reshape(1, H), p["W2"], p["b2"].reshape(1, H),
      lnp["g"].reshape(1, H), lnp["b"].reshape(1, H))


def _tc_update(h, aggs, cnts, wls, wr_sum_eye, bias_sum, g, b, want_table):
    n = h.shape[0]
    k_num = len(aggs)
    grid = pl.cdiv(n, BLK)

    def body(*refs):
        h_r = refs[0]
        agg_rs = refs[1:1 + k_num]
        cnt_rs = refs[1 + k_num:1 + 2 * k_num]
        wl_r = refs[1 + 2 * k_num]
        wr_r, bias_r, g_r, b_r = refs[2 + 2 * k_num:6 + 2 * k_num]
        o_rs = refs[6 + 2 * k_num:]
        u = jnp.dot(h_r[...], wr_r[...], preferred_element_type=jnp.float32)
        u = u + bias_r[...]
        means = []
        for k in range(k_num):
            c = cnt_rs[k]
            cs = c[0, :, 0:1] + c[1, :, 0:1]
            inv = 1.0 / jnp.maximum(cs, 1.0)
            means.append(agg_rs[k][...] * inv)
        mcat = jnp.concatenate(means, axis=1) if k_num > 1 else means[0]
        u = u + jnp.dot(mcat, wl_r[...], preferred_element_type=jnp.float32)
        hh = _layernorm_relu(u, g_r[...], b_r[...], relu=True)
        o_rs[0][...] = hh
        if want_table:
            for ch in range(NCHUNK):
                o_rs[1][ch] = hh[:, ch * FC:(ch + 1) * FC]

    in_specs = [pl.BlockSpec((BLK, H), lambda i: (i, 0))]
    in_specs += [pl.BlockSpec((BLK, H), lambda i: (i, 0))
                 for _ in range(k_num)]
    in_specs += [pl.BlockSpec((NC, BLK, CW), lambda i: (0, i, 0))
                 for _ in range(k_num)]
    in_specs += [
        pl.BlockSpec((k_num * H, H), lambda i: (0, 0)),
        pl.BlockSpec((H, H), lambda i: (0, 0)),
        pl.BlockSpec((1, H), lambda i: (0, 0)),
        pl.BlockSpec((1, H), lambda i: (0, 0)),
        pl.BlockSpec((1, H), lambda i: (0, 0)),
    ]
    out_specs = [pl.BlockSpec((BLK, H), lambda i: (i, 0))]
    out_shape = [jax.ShapeDtypeStruct((n, H), jnp.float32)]
    if want_table:
        out_specs.append(pl.BlockSpec((NCHUNK, BLK, FC), lambda i: (0, i, 0)))
        out_shape.append(jax.ShapeDtypeStruct((NCHUNK, n, FC), jnp.float32))

    wl_cat = jnp.concatenate(wls, axis=0) if k_num > 1 else wls[0]
    outs = pl.pallas_call(
        body, grid=(grid,), in_specs=in_specs, out_specs=out_specs,
        out_shape=out_shape,
    )(h, *aggs, *cnts, wl_cat, wr_sum_eye, bias_sum.reshape(1, H),
      g.reshape(1, H), b.reshape(1, H))
    return (outs[0], outs[1]) if want_table else (outs[0], None)


def kernel(x, params, edges):
    node_types = list(x.keys())
    nt_index = {nt: i for i, nt in enumerate(node_types)}
    n_nodes = {nt: x[nt].shape[0] for nt in node_types}
    n_pad = {nt: _round_up(n_nodes[nt], NS * 8) for nt in node_types}
    max_pad = max(n_pad.values())

    h, tables = {}, {}
    for nt in node_types:
        h[nt], tab = _tc_encoder(x[nt], params["enc"][nt],
                                 params["enc_ln"][nt])
        tables[nt] = tab.reshape(NCHUNK * n_nodes[nt], FC)

    ek_meta = []
    ek_names = []
    for ek, ei in edges.items():
        src_t, _, dst_t = ek.split("__")
        e_num = ei.shape[1]
        e_pad = _round_up(e_num, NS * W)
        rows = e_pad // W
        src = jnp.concatenate(
            [ei[0], jnp.zeros((e_pad + PADR * W - e_num,), jnp.int32)])
        dst = jnp.concatenate(
            [ei[1],
             jnp.full((e_pad + PADR * W - e_num,), n_nodes[dst_t],
                      jnp.int32)])
        offs = (jnp.arange(NCHUNK, dtype=jnp.int32) * n_nodes[src_t])
        srcA = (src[None, :] + offs[:, None]).reshape(NCHUNK, rows + PADR, W)
        dst2 = dst.reshape(rows + PADR, W)
        ek_meta.append({
            "srcA": srcA, "dst2": dst2, "rows": rows,
            "n_dst_pad": n_pad[dst_t], "tbl": nt_index[src_t],
            "src_t": src_t, "dst_t": dst_t,
        })
        ek_names.append(ek)

    zeros_f = jnp.zeros((max_pad, FC), jnp.float32)
    zeros_c = jnp.zeros((max_pad, CW), jnp.float32)
    ones = jnp.ones((W, CW), jnp.float32)
    cnts = _sc_count_merged(ek_meta, zeros_c, ones, max_pad)

    for li, layer in enumerate(params["layers"]):
        want_table = li + 1 < len(params["layers"])
        tables_list = [tables[nt] for nt in node_types]
        aggs = _sc_agg_layer(tables_list, ek_meta, zeros_f, max_pad)
        gathered = {nt: [] for nt in node_types}
        for i, ek in enumerate(ek_names):
            dst_t = ek_meta[i]["dst_t"]
            gathered[dst_t].append((aggs[i], cnts[i], layer["conv"][ek]))
        new_h, new_tables = {}, {}
        for nt in node_types:
            parts = gathered[nt]
            ag = [p[0] for p in parts]
            cn = [p[1] for p in parts]
            wls = [p[2]["Wl"] for p in parts]
            wr_sum = sum(p[2]["Wr"] for p in parts) + jnp.eye(
                H, dtype=jnp.float32)
            bias_sum = sum(p[2]["bl"] for p in parts)
            ln = layer["ln"][nt]
            new_h[nt], tab = _tc_update(h[nt], ag, cn, wls, wr_sum,
                                        bias_sum, ln["g"], ln["b"],
                                        want_table)
            if want_table:
                new_tables[nt] = tab.reshape(NCHUNK * n_nodes[nt], FC)
        h = new_h
        if want_table:
            tables = new_tables

    return (h["note"], h["chord"], h["onset"])


# speedup vs baseline: 1.3478x; 1.0247x over previous
import functools

import jax
import jax.numpy as jnp
from jax import lax
from jax.experimental import pallas as pl
from jax.experimental.pallas import tpu as pltpu
from jax.experimental.pallas import tpu_sc as plsc

NC = 2
NS = 16
NW = NC * NS
W = 128
H = 128
FC = 32
NCHUNK = H // FC
CW = 16
BLK = 512
NBUF = 6
PADR = NW + NBUF + 4


def _round_up(a, m):
    return (a + m - 1) // m * m


def _sc_mesh():
    return plsc.VectorSubcoreMesh(core_axis_name="c", subcore_axis_name="s")


def _sc_agg_layer(tables_list, ek_meta, zeros, max_pad):
    n_ek = len(ek_meta)
    n_nt = len(tables_list)
    out_types = tuple(
        jax.ShapeDtypeStruct((m["n_dst_pad"], H), jnp.float32)
        for m in ek_meta)
    inputs = list(tables_list)
    for m in ek_meta:
        inputs += [m["srcA"], m["dst2"]]
    inputs.append(zeros)

    @functools.partial(
        pl.kernel,
        mesh=_sc_mesh(),
        out_type=out_types,
        compiler_params=pltpu.CompilerParams(use_tc_tiling_on_sc=False),
        scratch_types=[
            pltpu.VMEM((NBUF, W), jnp.int32),
            pltpu.VMEM((NBUF, W), jnp.int32),
            pltpu.VMEM((NBUF, W), jnp.int32),
            pltpu.VMEM((NBUF, W), jnp.int32),
            [pltpu.VMEM((W, FC), jnp.float32) for _ in range(NBUF)],
            pltpu.VMEM_SHARED((max_pad, FC), jnp.float32),
            [pltpu.SemaphoreType.DMA for _ in range(NBUF)],
            [pltpu.SemaphoreType.DMA for _ in range(NBUF)],
            pltpu.SemaphoreType.DMA,
            pltpu.SemaphoreType.DMA,
        ],
    )
    def k(*args):
        nt_refs = args[:n_nt]
        ek_refs = [(args[n_nt + 2 * i], args[n_nt + 2 * i + 1])
                   for i in range(n_ek)]
        zero_h = args[n_nt + 2 * n_ek]
        out_refs = args[n_nt + 2 * n_ek + 1:n_nt + 2 * n_ek + 1 + n_ek]
        (st_as, st_ad, st_bs, st_bd, rows, acc, g_sem, s_sem, sem_a,
         sem_b) = args[n_nt + 2 * n_ek + 1 + n_ek:]
        cid = lax.axis_index("c")
        sid = lax.axis_index("s")
        stages = ((st_as, st_ad, sem_a), (st_bs, st_bd, sem_b))

        def phase(table_h, srcA_h, dst_h, out_h, nwin, n_dst_pad, sub):
            n_grp = -(-nwin // NBUF)
            n_sup = n_grp // 2
            rpt = n_dst_pad // NS
            r0 = sid * rpt
            chunk = 2 * cid + sub
            w0 = sid * nwin

            def stage_load(g, st):
                pltpu.async_copy(
                    srcA_h.at[chunk, pl.ds(w0 + g * NBUF, NBUF)], st[0],
                    st[2])
                pltpu.async_copy(
                    dst_h.at[pl.ds(w0 + g * NBUF, NBUF)], st[1], st[2])

            def stage_wait(g, st):
                pltpu.make_async_copy(
                    srcA_h.at[chunk, pl.ds(w0 + g * NBUF, NBUF)], st[0],
                    st[2]).wait()
                pltpu.make_async_copy(
                    dst_h.at[pl.ds(w0 + g * NBUF, NBUF)], st[1],
                    st[2]).wait()

            def g_start(b, st):
                pltpu.async_copy(table_h.at[st[0].at[b]], rows[b], g_sem[b])

            def g_wait(b, st):
                pltpu.make_async_copy(table_h.at[st[0].at[b]], rows[b],
                                      g_sem[b]).wait()

            def s_start(b, st):
                pltpu.async_copy(rows[b], acc.at[st[1].at[b]], s_sem[b],
                                 add=True)

            def s_wait(b, st):
                pltpu.make_async_copy(rows[b], acc.at[st[1].at[b]],
                                      s_sem[b]).wait()

            pltpu.sync_copy(zero_h.at[pl.ds(r0, rpt)], acc.at[pl.ds(r0, rpt)])
            plsc.subcore_barrier()

            stage_load(0, stages[0])
            stage_wait(0, stages[0])
            for b in range(min(NBUF, nwin)):
                g_start(b, stages[0])

            def grp(gprev, gnext, cst, nst):
                for b in range(NBUF):
                    @pl.when(gprev * NBUF + b < nwin)
                    def _(b=b):
                        g_wait(b, cst)
                        s_start(b, cst)

                @pl.when(gnext * NBUF < nwin)
                def _():
                    stage_wait(gnext, nst)

                for b in range(NBUF):
                    @pl.when(gnext * NBUF + b < nwin)
                    def _(b=b):
                        s_wait(b, cst)
                        g_start(b, nst)

            @pl.loop(0, n_sup)
            def _sup(gg):
                g0 = 2 * gg
                g1 = g0 + 1

                @pl.when(g1 * NBUF < nwin)
                def _():
                    stage_load(g1, stages[1])

                grp(g0, g1, stages[0], stages[1])

                @pl.when((g1 + 1) * NBUF < nwin)
                def _():
                    stage_load(g1 + 1, stages[0])

                grp(g1, g1 + 1, stages[1], stages[0])

            if n_grp % 2 == 1:
                gl = n_grp - 1
                lst = stages[gl % 2]
                for b in range(NBUF):
                    if gl * NBUF + b < nwin:
                        g_wait(b, lst)
                        s_start(b, lst)
            for b in range(min(NBUF, nwin)):
                s_wait(b, stages[0])

            plsc.subcore_barrier()
            pltpu.sync_copy(acc.at[pl.ds(r0, rpt)],
                            out_h.at[pl.ds(r0, rpt), pl.ds(chunk * FC, FC)])
            plsc.subcore_barrier()

        for sub in range(2):
            for i, m in enumerate(ek_meta):
                phase(nt_refs[m["tbl"]], ek_refs[i][0], ek_refs[i][1],
                      out_refs[i], m["rows"] // NS, m["n_dst_pad"], sub)

    return k(*inputs)


def _sc_count_merged(ek_meta, zeros16, ones, max_pad):
    n_ek = len(ek_meta)
    out_types = tuple(
        jax.ShapeDtypeStruct((NC, m["n_dst_pad"], CW), jnp.float32)
        for m in ek_meta)
    inputs = [m["dst2"] for m in ek_meta] + [zeros16, ones]

    @functools.partial(
        pl.kernel,
        mesh=_sc_mesh(),
        out_type=out_types,
        compiler_params=pltpu.CompilerParams(use_tc_tiling_on_sc=False),
        scratch_types=[
            pltpu.VMEM((W,), jnp.int32),
            pltpu.VMEM((W, CW), jnp.float32),
            pltpu.VMEM_SHARED((max_pad, CW), jnp.float32),
            pltpu.SemaphoreType.DMA,
        ],
    )
    def k(*args):
        dst_refs = args[:n_ek]
        zero_h, ones_h = args[n_ek], args[n_ek + 1]
        out_refs = args[n_ek + 2:2 * n_ek + 2]
        idx_d, ones_v, acc, sem = args[2 * n_ek + 2:]
        cid = lax.axis_index("c")
        sid = lax.axis_index("s")
        wid = sid * NC + cid
        pltpu.sync_copy(ones_h, ones_v)

        for i, m in enumerate(ek_meta):
            dst_h, out_h = dst_refs[i], out_refs[i]
            rows = m["rows"]
            nwin = -(-rows // NW)
            rpt = m["n_dst_pad"] // NS
            r0 = sid * rpt
            pltpu.sync_copy(zero_h.at[pl.ds(r0, rpt)], acc.at[pl.ds(r0, rpt)])
            plsc.subcore_barrier()

            @pl.loop(0, nwin)
            def _win(win, dst_h=dst_h, nwin=nwin, rows=rows):
                w = wid * nwin + win

                @pl.when(w < rows)
                def _():
                    pltpu.sync_copy(dst_h.at[w], idx_d)
                    pltpu.sync_copy(ones_v, acc.at[idx_d], add=True)

            plsc.subcore_barrier()
            pltpu.sync_copy(acc.at[pl.ds(r0, rpt)],
                            out_h.at[cid, pl.ds(r0, rpt)])
            plsc.subcore_barrier()

    return k(*inputs)


def _layernorm_relu(e, g, b, relu):
    mu = jnp.mean(e, axis=1, keepdims=True)
    var = jnp.mean(jnp.square(e - mu), axis=1, keepdims=True)
    out = (e - mu) * lax.rsqrt(var + 1e-5) * g + b
    return jnp.maximum(out, 0.0) if relu else out


def _tc_encoder(xin, p, lnp):
    n, d = xin.shape
    grid = pl.cdiv(n, BLK)

    def body(x_r, w1_r, b1_r, w2_r, b2_r, g_r, bb_r, o_r, oc_r):
        h1 = jnp.dot(x_r[...], w1_r[...], preferred_element_type=jnp.float32)
        h1 = jnp.maximum(h1 + b1_r[...], 0.0)
        e = jnp.dot(h1, w2_r[...], preferred_element_type=jnp.float32)
        e = e + b2_r[...]
        hh = _layernorm_relu(e, g_r[...], bb_r[...], relu=False)
        o_r[...] = hh
        for c in range(NCHUNK):
            oc_r[c] = hh[:, c * FC:(c + 1) * FC]

    return pl.pallas_call(
        body,
        grid=(grid,),
        in_specs=[
            pl.BlockSpec((BLK, d), lambda i: (i, 0)),
            pl.BlockSpec((d, H), lambda i: (0, 0)),
            pl.BlockSpec((1, H), lambda i: (0, 0)),
            pl.BlockSpec((H, H), lambda i: (0, 0)),
            pl.BlockSpec((1, H), lambda i: (0, 0)),
            pl.BlockSpec((1, H), lambda i: (0, 0)),
            pl.BlockSpec((1, H), lambda i: (0, 0)),
        ],
        out_specs=[
            pl.BlockSpec((BLK, H), lambda i: (i, 0)),
            pl.BlockSpec((NCHUNK, BLK, FC), lambda i: (0, i, 0)),
        ],
        out_shape=[
            jax.ShapeDtypeStruct((n, H), jnp.float32),
            jax.ShapeDtypeStruct((NCHUNK, n, FC), jnp.float32),
        ],
    )(xin, p["W1"], p["b1"].reshape(1, H), p["W2"], p["b2"].reshape(1, H),
      lnp["g"].reshape(1, H), lnp["b"].reshape(1, H))


def _tc_update(h, aggs, cnts, wls, wr_sum_eye, bias_sum, g, b, want_table):
    n = h.shape[0]
    k_num = len(aggs)
    grid = pl.cdiv(n, BLK)

    def body(*refs):
        h_r = refs[0]
        agg_rs = refs[1:1 + k_num]
        cnt_rs = refs[1 + k_num:1 + 2 * k_num]
        wl_r = refs[1 + 2 * k_num]
        wr_r, bias_r, g_r, b_r = refs[2 + 2 * k_num:6 + 2 * k_num]
        o_rs = refs[6 + 2 * k_num:]
        u = jnp.dot(h_r[...], wr_r[...], preferred_element_type=jnp.float32)
        u = u + bias_r[...]
        means = []
        for k in range(k_num):
            c = cnt_rs[k]
            cs = c[0, :, 0:1] + c[1, :, 0:1]
            inv = 1.0 / jnp.maximum(cs, 1.0)
            means.append(agg_rs[k][...] * inv)
        mcat = jnp.concatenate(means, axis=1) if k_num > 1 else means[0]
        u = u + jnp.dot(mcat, wl_r[...], preferred_element_type=jnp.float32)
        hh = _layernorm_relu(u, g_r[...], b_r[...], relu=True)
        o_rs[0][...] = hh
        if want_table:
            for ch in range(NCHUNK):
                o_rs[1][ch] = hh[:, ch * FC:(ch + 1) * FC]

    in_specs = [pl.BlockSpec((BLK, H), lambda i: (i, 0))]
    in_specs += [pl.BlockSpec((BLK, H), lambda i: (i, 0))
                 for _ in range(k_num)]
    in_specs += [pl.BlockSpec((NC, BLK, CW), lambda i: (0, i, 0))
                 for _ in range(k_num)]
    in_specs += [
        pl.BlockSpec((k_num * H, H), lambda i: (0, 0)),
        pl.BlockSpec((H, H), lambda i: (0, 0)),
        pl.BlockSpec((1, H), lambda i: (0, 0)),
        pl.BlockSpec((1, H), lambda i: (0, 0)),
        pl.BlockSpec((1, H), lambda i: (0, 0)),
    ]
    out_specs = [pl.BlockSpec((BLK, H), lambda i: (i, 0))]
    out_shape = [jax.ShapeDtypeStruct((n, H), jnp.float32)]
    if want_table:
        out_specs.append(pl.BlockSpec((NCHUNK, BLK, FC), lambda i: (0, i, 0)))
        out_shape.append(jax.ShapeDtypeStruct((NCHUNK, n, FC), jnp.float32))

    wl_cat = jnp.concatenate(wls, axis=0) if k_num > 1 else wls[0]
    outs = pl.pallas_call(
        body, grid=(grid,), in_specs=in_specs, out_specs=out_specs,
        out_shape=out_shape,
    )(h, *aggs, *cnts, wl_cat, wr_sum_eye, bias_sum.reshape(1, H),
      g.reshape(1, H), b.reshape(1, H))
    return (outs[0], outs[1]) if want_table else (outs[0], None)


def kernel(x, params, edges):
    node_types = list(x.keys())
    nt_index = {nt: i for i, nt in enumerate(node_types)}
    n_nodes = {nt: x[nt].shape[0] for nt in node_types}
    n_pad = {nt: _round_up(n_nodes[nt], NS * 8) for nt in node_types}
    max_pad = max(n_pad.values())

    h, tables = {}, {}
    for nt in node_types:
        h[nt], tab = _tc_encoder(x[nt], params["enc"][nt],
                                 params["enc_ln"][nt])
        tables[nt] = tab.reshape(NCHUNK * n_nodes[nt], FC)

    ek_meta = []
    ek_names = []
    for ek, ei in edges.items():
        src_t, _, dst_t = ek.split("__")
        e_num = ei.shape[1]
        e_pad = _round_up(e_num, NS * W)
        rows = e_pad // W
        src = jnp.concatenate(
            [ei[0], jnp.zeros((e_pad + PADR * W - e_num,), jnp.int32)])
        dst = jnp.concatenate(
            [ei[1],
             jnp.full((e_pad + PADR * W - e_num,), n_nodes[dst_t],
                      jnp.int32)])
        offs = (jnp.arange(NCHUNK, dtype=jnp.int32) * n_nodes[src_t])
        srcA = (src[None, :] + offs[:, None]).reshape(NCHUNK, rows + PADR, W)
        dst2 = dst.reshape(rows + PADR, W)
        ek_meta.append({
            "srcA": srcA, "dst2": dst2, "rows": rows,
            "n_dst_pad": n_pad[dst_t], "tbl": nt_index[src_t],
            "src_t": src_t, "dst_t": dst_t,
        })
        ek_names.append(ek)

    zeros_f = jnp.zeros((max_pad, FC), jnp.float32)
    zeros_c = jnp.zeros((max_pad, CW), jnp.float32)
    ones = jnp.ones((W, CW), jnp.float32)
    cnts = _sc_count_merged(ek_meta, zeros_c, ones, max_pad)

    for li, layer in enumerate(params["layers"]):
        want_table = li + 1 < len(params["layers"])
        tables_list = [tables[nt] for nt in node_types]
        aggs = _sc_agg_layer(tables_list, ek_meta, zeros_f, max_pad)
        gathered = {nt: [] for nt in node_types}
        for i, ek in enumerate(ek_names):
            dst_t = ek_meta[i]["dst_t"]
            gathered[dst_t].append((aggs[i], cnts[i], layer["conv"][ek]))
        new_h, new_tables = {}, {}
        for nt in node_types:
            parts = gathered[nt]
            ag = [p[0] for p in parts]
            cn = [p[1] for p in parts]
            wls = [p[2]["Wl"] for p in parts]
            wr_sum = sum(p[2]["Wr"] for p in parts) + jnp.eye(
                H, dtype=jnp.float32)
            bias_sum = sum(p[2]["bl"] for p in parts)
            ln = layer["ln"][nt]
            new_h[nt], tab = _tc_update(h[nt], ag, cn, wls, wr_sum,
                                        bias_sum, ln["g"], ln["b"],
                                        want_table)
            if want_table:
                new_tables[nt] = tab.reshape(NCHUNK * n_nodes[nt], FC)
        h = new_h
        if want_table:
            tables = new_tables

    return (h["note"], h["chord"], h["onset"])
